# baseline (speedup 1.0000x reference)
import functools

import jax
import jax.numpy as jnp
from jax import lax
from jax.experimental import pallas as pl
from jax.experimental.pallas import tpu as pltpu
from jax.experimental.pallas import tpu_sc as plsc
from jax.experimental.layout import Layout, with_layout_constraint

_NC, _NS = 2, 16
_NW = _NC * _NS
_CHUNK = 104


def kernel(x, weight):
    batch, n_fields = x.shape
    _, d = weight.shape
    n = batch * n_fields
    idx2 = x.reshape(n // 32, 32).astype(jnp.int32) * 4
    rep2 = jnp.take(idx2, jnp.arange(128) // 4, axis=1)
    rep2 = rep2 + (jnp.arange(128, dtype=jnp.int32) % 4) * 4
    idx_rep = rep2.reshape(n * 4)
    w_sc = with_layout_constraint(
        weight, Layout(major_to_minor=(0, 1), tiling=((8,),))
    )
    b_per_w = n // _NW
    n_chunks = b_per_w // _CHUNK
    crep = _CHUNK * 4

    mesh = plsc.VectorSubcoreMesh(core_axis_name="c", subcore_axis_name="s")

    rows_per_chunk = _CHUNK // n_fields

    @functools.partial(
        pl.kernel,
        mesh=mesh,
        out_type=jax.ShapeDtypeStruct((batch, n_fields, d), jnp.float32),
        scratch_types=[
            pltpu.VMEM((crep,), jnp.int32),
            pltpu.VMEM((crep,), jnp.int32),
            pltpu.VMEM((crep, d), jnp.float32),
            pltpu.VMEM((crep, d), jnp.float32),
            pltpu.SemaphoreType.DMA,
            pltpu.SemaphoreType.DMA,
            pltpu.SemaphoreType.DMA,
            pltpu.SemaphoreType.DMA,
        ],
    )
    def k(
        table_hbm,
        idx_hbm,
        out_hbm,
        idx_v0,
        idx_v1,
        rows_v0,
        rows_v1,
        sg0,
        sg1,
        so0,
        so1,
    ):
        wid = lax.axis_index("s") * _NC + lax.axis_index("c")
        wbase = wid * b_per_w

        def out_copy(rows_v, chunk, sem):
            return pltpu.async_copy(
                rows_v.at[pl.ds(0, _CHUNK)].reshape(
                    rows_per_chunk, n_fields, d
                ),
                out_hbm.at[
                    pl.ds(
                        (wbase + chunk * _CHUNK) // n_fields, rows_per_chunk
                    )
                ],
                sem,
            )

        def idx_load(idx_v, chunk):
            chunk = jnp.minimum(chunk, n_chunks - 1)
            pltpu.sync_copy(
                idx_hbm.at[pl.ds((wbase + chunk * _CHUNK) * 4, crep)], idx_v
            )

        idx_load(idx_v0, 0)

        @pl.loop(0, n_chunks, step=2)
        def _(t):
            ga = pltpu.async_copy(table_hbm.at[idx_v0], rows_v0, sg0)
            idx_load(idx_v1, t + 1)
            ga.wait()
            oa = out_copy(rows_v0, t, so0)
            gb = pltpu.async_copy(table_hbm.at[idx_v1], rows_v1, sg1)
            idx_load(idx_v0, t + 2)
            gb.wait()
            ob = out_copy(rows_v1, t + 1, so1)
            oa.wait()
            ob.wait()

    out = k(w_sc, idx_rep)
    return out


# speedup vs baseline: 1.3134x; 1.3134x over previous
import functools

import jax
import jax.numpy as jnp
from jax import lax
from jax.experimental import pallas as pl
from jax.experimental.pallas import tpu as pltpu
from jax.experimental.pallas import tpu_sc as plsc
from jax.experimental.layout import Layout, with_layout_constraint

_NC, _NS = 2, 16
_NW = _NC * _NS
_CHUNK = 104


def kernel(x, weight):
    batch, n_fields = x.shape
    _, d = weight.shape
    n = batch * n_fields
    idx2 = x.reshape(n // 32, 32).astype(jnp.int32) * 4
    rep2 = jnp.take(idx2, jnp.arange(128) // 4, axis=1)
    rep2 = rep2 + (jnp.arange(128, dtype=jnp.int32) % 4) * 4
    idx_rep = rep2.reshape(n * 4)
    w_sc = with_layout_constraint(
        weight, Layout(major_to_minor=(0, 1), tiling=((8,),))
    )
    b_per_w = n // _NW
    n_chunks = b_per_w // _CHUNK
    crep = _CHUNK * 4

    mesh = plsc.VectorSubcoreMesh(core_axis_name="c", subcore_axis_name="s")

    rows_per_chunk = _CHUNK // n_fields

    @functools.partial(
        pl.kernel,
        mesh=mesh,
        out_type=jax.ShapeDtypeStruct((batch, n_fields, d), jnp.float32),
        scratch_types=[
            pltpu.VMEM((crep,), jnp.int32),
            pltpu.VMEM((crep,), jnp.int32),
            pltpu.VMEM((crep, d), jnp.float32),
            pltpu.VMEM((crep, d), jnp.float32),
            pltpu.SemaphoreType.DMA,
            pltpu.SemaphoreType.DMA,
            pltpu.SemaphoreType.DMA,
            pltpu.SemaphoreType.DMA,
        ],
    )
    def k(
        table_hbm,
        idx_hbm,
        out_hbm,
        idx_v0,
        idx_v1,
        rows_v0,
        rows_v1,
        sg0,
        sg1,
        so0,
        so1,
    ):
        wid = lax.axis_index("s") * _NC + lax.axis_index("c")
        wbase = wid * b_per_w

        def out_copy(rows_v, chunk, sem):
            return pltpu.async_copy(
                rows_v.at[pl.ds(0, _CHUNK)].reshape(
                    rows_per_chunk, n_fields, d
                ),
                out_hbm.at[
                    pl.ds(
                        (wbase + chunk * _CHUNK) // n_fields, rows_per_chunk
                    )
                ],
                sem,
            )

        def idx_load(idx_v, chunk):
            chunk = jnp.minimum(chunk, n_chunks - 1)
            pltpu.sync_copy(
                idx_hbm.at[pl.ds((wbase + chunk * _CHUNK) * 4, crep)], idx_v
            )

        idx_load(idx_v0, 0)

        @pl.loop(0, n_chunks, step=2)
        def _(t):
            ga = pltpu.async_copy(table_hbm.at[idx_v0], rows_v0, sg0)
            idx_load(idx_v1, t + 1)
            ga.wait()
            oa = out_copy(rows_v0, t, so0)
            gb = pltpu.async_copy(table_hbm.at[idx_v1], rows_v1, sg1)
            idx_load(idx_v0, t + 2)
            gb.wait()
            ob = out_copy(rows_v1, t + 1, so1)
            oa.wait()
            ob.wait()

    out = k(w_sc, idx_rep)
    out = with_layout_constraint(
        out, Layout(major_to_minor=(0, 1, 2), tiling=((8, 128),))
    )
    return out
